# baseline (speedup 1.0000x reference)
import functools

import jax
import jax.numpy as jnp
from jax import lax
from jax.experimental import pallas as pl
from jax.experimental.pallas import tpu as pltpu


def _mlp_kernel(nk, nt, x_ref, wa_ref, wb_ref, m_ref, b0_ref,
                w1_ref, m1_ref, b1_ref, w2_ref, m2_ref, b2_ref,
                o_ref, acc_ref, acc1_ref):
    k = pl.program_id(0)

    @pl.when(k == 0)
    def _():
        acc_ref[...] = jnp.zeros_like(acc_ref)

    @pl.when(k < nk)
    def _():
        xb = x_ref[...].astype(jnp.bfloat16)
        half = wa_ref.shape[0]
        ng = 2 * half // min(512, half)
        grp = 2 * half // ng
        for g in range(ng):
            w_ref = wa_ref if g < ng // 2 else wb_ref
            wsl = pl.ds((g % (ng // 2)) * grp, grp)
            sl = pl.ds(g * grp, grp)
            wg = jnp.where(m_ref[sl, :] != 0, w_ref[wsl, :],
                           0.0).astype(jnp.bfloat16)
            pg = lax.dot_general(xb, wg, (((1,), (1,)), ((), ())),
                                 preferred_element_type=jnp.float32)
            acc_ref[:, sl] += pg

    @pl.when(k >= nk)
    def _():
        kk = k - nk
        bt = w1_ref.shape[1]
        hb = jnp.maximum(acc_ref[:, pl.ds(kk * bt, bt)] + b0_ref[...],
                         0.0).astype(jnp.bfloat16)
        w1b = jnp.where(m1_ref[...] != 0, w1_ref[...],
                        0.0).astype(jnp.bfloat16)
        part = lax.dot_general(hb, w1b, (((1,), (1,)), ((), ())),
                               preferred_element_type=jnp.float32)

        @pl.when(kk == 0)
        def _():
            acc1_ref[...] = part

        @pl.when(kk > 0)
        def _():
            acc1_ref[...] += part

        @pl.when(kk == nt - 1)
        def _():
            h1 = jnp.maximum(acc1_ref[...] + b1_ref[...],
                             0.0).astype(jnp.bfloat16)
            w2b = jnp.where(m2_ref[...] != 0, w2_ref[...],
                            0.0).astype(jnp.bfloat16)
            out = lax.dot_general(h1, w2b, (((1,), (1,)), ((), ())),
                                  preferred_element_type=jnp.float32)
            o_ref[...] = out + b2_ref[...]


@functools.partial(jax.jit, static_argnames=("block_k", "block_t"))
def _masked_mlp(x, W0, b0, W1, b1, W2, b2, mask0, mask1, mask2,
                block_k=512, block_t=512):
    B, K0 = x.shape
    N0 = W0.shape[0]
    N1 = W1.shape[0]
    N2 = W2.shape[0]
    bk = min(block_k, K0)
    nk = K0 // bk
    bt = min(block_t, N0)
    nt = N0 // bt

    m0 = mask0.view(jnp.uint8)
    m1 = mask1.view(jnp.uint8)
    m2 = mask2.view(jnp.uint8)

    kmax = nk - 1
    full = lambda *s: pl.BlockSpec(s, lambda i: tuple(0 for _ in s))
    return pl.pallas_call(
        functools.partial(_mlp_kernel, nk, nt),
        grid=(nk + nt,),
        in_specs=[
            pl.BlockSpec((B, bk), lambda i: (0, jnp.minimum(i, kmax))),
            pl.BlockSpec((N0 // 2, bk),
                         lambda i: (0, jnp.minimum(i, kmax))),
            pl.BlockSpec((N0 // 2, bk),
                         lambda i: (1, jnp.minimum(i, kmax))),
            pl.BlockSpec((N0, bk), lambda i: (0, jnp.minimum(i, kmax))),
            pl.BlockSpec((1, bt), lambda i: (0, jnp.clip(i - nk, 0, nt - 1))),
            pl.BlockSpec((N1, bt),
                         lambda i: (0, jnp.clip(i - nk, 0, nt - 1))),
            pl.BlockSpec((N1, bt),
                         lambda i: (0, jnp.clip(i - nk, 0, nt - 1))),
            full(1, N1),
            full(N2, N1), full(N2, N1), full(1, N2),
        ],
        out_specs=full(B, N2),
        out_shape=jax.ShapeDtypeStruct((B, N2), jnp.float32),
        scratch_shapes=[pltpu.VMEM((B, N0), jnp.float32),
                        pltpu.VMEM((B, N1), jnp.float32)],
        compiler_params=pltpu.CompilerParams(
            dimension_semantics=("arbitrary",)),
    )(x, W0, W0, m0, b0.reshape(1, -1), W1, m1, b1.reshape(1, -1),
      W2, m2, b2.reshape(1, -1))


def kernel(x, W0, b0, W1, b1, W2, b2, mask0, mask1, mask2):
    return _masked_mlp(x, W0, b0, W1, b1, W2, b2, mask0, mask1, mask2)
